# baseline (speedup 1.0000x reference)
import functools

import jax
import jax.numpy as jnp
from jax import lax
from jax.experimental import pallas as pl
from jax.experimental.pallas import tpu as pltpu
from jax.experimental.pallas import tpu_sc as plsc

_B = 16384
_D = 64
_V = 100000
_NC = 2
_NS = 16
_NW = _NC * _NS
_W = 128
_NWIN_FULL = _V // _W
_TAIL = _V - _NWIN_FULL * _W
_TAIL_WORKER = _NWIN_FULL % _NW
_TAIL_J = _NWIN_FULL // _NW
_UVI_ROWS = _B + 128
_OUT_ROWS = _B + 16

_i32 = jnp.int32
_f32 = jnp.float32


def _worker_id():
    return lax.axis_index("s") * _NC + lax.axis_index("c")


def _scan_ids(w, idsv, mlist, lane):
    def body(i, cnt):
        ids = idsv[pl.ds(i * 16, 16)]
        k_all = lax.shift_right_logical(ids, 7)
        mine = (k_all & 31) == w
        j = lax.shift_right_logical(k_all, 5)
        col = ids & 127
        b = i * 16 + lane
        e = (b << 12) | (j << 7) | col
        plsc.store_compressed(mlist.at[pl.ds(cnt, 16)], e, mask=mine)
        return cnt + plsc.all_reduce_population_count(mine)[0]

    return lax.fori_loop(0, _B // 16, body, 0)


def _collect_window(j, cnt, mlist, wlist, lane):
    def body(ci, wcnt):
        e = mlist[pl.ds(ci * 16, 16)]
        sel = ((lax.shift_right_logical(e, 7) & 31) == j) \
            & ((ci * 16 + lane) < cnt)
        plsc.store_compressed(wlist.at[pl.ds(wcnt, 16)], e, mask=sel)
        return wcnt + plsc.all_reduce_population_count(sel)[0]

    nmc = lax.shift_right_logical(cnt + 15, 4)
    return lax.fori_loop(0, nmc, body, 0)


def _init_dummy_rows(bst, base):
    lane = lax.iota(_i32, 16)
    for m in range(8):
        bst[pl.ds(m * 16, 16)] = base + m * 16 + lane


def _tower_a(utT, utTt, uids, uvi, idsv, mlist, wlist, win,
             sbuf, bst, fillref):
    w = _worker_id()
    lane = lax.iota(_i32, 16)
    pltpu.sync_copy(uids, idsv)
    cnt = _scan_ids(w, idsv, mlist, lane)

    fillref[0] = 0
    _init_dummy_rows(bst, _B)

    def flush():
        pltpu.sync_copy(sbuf, uvi.at[bst])
        _init_dummy_rows(bst, _B)
        fillref[0] = 0

    def extract(j, winref, wcnt):
        def body(ci, carry):
            base = ci * 16
            e = wlist[pl.ds(base, 16)]
            sel = lane < (wcnt - base)
            cols = e & 127
            bs = lax.shift_right_logical(e, 12)

            @pl.when(fillref[0] + 16 > 128)
            def _():
                flush()

            fill = fillref[0]
            slot = fill + plsc.cumsum(sel.astype(_i32)) - 1
            for d in range(_D):
                dv = jnp.full((16,), d, _i32)
                vals = plsc.load_gather(winref, [dv, cols], mask=sel)
                plsc.store_scatter(sbuf, [slot, dv], vals, mask=sel)
            plsc.store_scatter(bst, [slot], bs, mask=sel)
            fillref[0] = fill + plsc.all_reduce_population_count(sel)[0]
            return carry

        nec = lax.shift_right_logical(wcnt + 15, 4)
        lax.fori_loop(0, nec, body, 0)

    nfull = jnp.where(w <= 12, 25, 24)

    def win_body(j, carry):
        k = w + 32 * j
        pltpu.sync_copy(utT.at[:, pl.ds(k * _W, _W)], win)
        wcnt = _collect_window(j, cnt, mlist, wlist, lane)
        extract(j, win, wcnt)
        return carry

    lax.fori_loop(0, nfull, win_body, 0)

    @pl.when(w == _TAIL_WORKER)
    def _():
        pltpu.sync_copy(utTt, win)
        wcnt = _collect_window(_TAIL_J, cnt, mlist, wlist, lane)
        extract(_TAIL_J, win, wcnt)

    @pl.when(fillref[0] > 0)
    def _():
        flush()


def _tower_b(itT, itTt, iids, uvi, outp, idsv, mlist, wlist, win,
             ubuf, bidx, ovals):
    w = _worker_id()
    lane = lax.iota(_i32, 16)
    pltpu.sync_copy(iids, idsv)
    cnt = _scan_ids(w, idsv, mlist, lane)

    def dot_chunks(j, winref, wcnt):
        def body(ci, carry):
            base = ci * 16
            e = wlist[pl.ds(base, 16)]
            sel = lane < (wcnt - base)
            cols = e & 127
            bs = lax.shift_right_logical(e, 12)
            m = sel.astype(_i32)
            bidx[pl.ds(0, 16)] = bs * m + (_B + lane) * (1 - m)
            pltpu.sync_copy(uvi.at[bidx.at[pl.ds(0, 16)]], ubuf)
            acc0 = jnp.zeros((16,), _f32)
            acc1 = jnp.zeros((16,), _f32)
            accs = [acc0, acc1]
            for d in range(_D):
                dv = jnp.full((16,), d, _i32)
                iv = plsc.load_gather(winref, [dv, cols])
                uv = plsc.load_gather(ubuf, [lane, dv])
                accs[d % 2] = accs[d % 2] + iv * uv
            ovals[pl.ds(0, 16)] = accs[0] + accs[1]
            pltpu.sync_copy(ovals.at[pl.ds(0, 16)],
                            outp.at[bidx.at[pl.ds(0, 16)]])
            return carry

        nec = lax.shift_right_logical(wcnt + 15, 4)
        lax.fori_loop(0, nec, body, 0)

    nfull = jnp.where(w <= 12, 25, 24)

    def win_body(j, carry):
        k = w + 32 * j
        pltpu.sync_copy(itT.at[:, pl.ds(k * _W, _W)], win)
        wcnt = _collect_window(j, cnt, mlist, wlist, lane)
        dot_chunks(j, win, wcnt)
        return carry

    lax.fori_loop(0, nfull, win_body, 0)

    @pl.when(w == _TAIL_WORKER)
    def _():
        pltpu.sync_copy(itTt, win)
        wcnt = _collect_window(_TAIL_J, cnt, mlist, wlist, lane)
        dot_chunks(_TAIL_J, win, wcnt)


@jax.jit
def _two_tower(uids, iids, utab, itab):
    mesh = plsc.VectorSubcoreMesh(core_axis_name="c", subcore_axis_name="s")
    cp = pltpu.CompilerParams(needs_layout_passes=False,
                              use_tc_tiling_on_sc=True)
    utT = utab.T
    itT = itab.T
    zpad = jnp.zeros((_W - _TAIL, _D), _f32)
    utTt = jnp.concatenate([utab[_NWIN_FULL * _W:], zpad]).T
    itTt = jnp.concatenate([itab[_NWIN_FULL * _W:], zpad]).T

    kern_a = functools.partial(
        pl.kernel, mesh=mesh,
        out_type=jax.ShapeDtypeStruct((_UVI_ROWS, 128), _f32),
        scratch_types=[
            pltpu.VMEM((_B,), _i32),
            pltpu.VMEM((_B,), _i32),
            pltpu.VMEM((_B,), _i32),
            pltpu.VMEM((_D, _W), _f32),
            pltpu.VMEM((128, 128), _f32),
            pltpu.VMEM((128,), _i32),
            pltpu.SMEM((1,), _i32),
        ],
        compiler_params=cp,
    )(_tower_a)
    uvi = kern_a(utT, utTt, uids)

    kern_b = functools.partial(
        pl.kernel, mesh=mesh,
        out_type=jax.ShapeDtypeStruct((_OUT_ROWS,), _f32),
        scratch_types=[
            pltpu.VMEM((_B,), _i32),
            pltpu.VMEM((_B,), _i32),
            pltpu.VMEM((_B,), _i32),
            pltpu.VMEM((_D, _W), _f32),
            pltpu.VMEM((16, 128), _f32),
            pltpu.VMEM((128,), _i32),
            pltpu.VMEM((128,), _f32),
        ],
        compiler_params=cp,
    )(_tower_b)
    outp = kern_b(itT, itTt, iids, uvi)
    return outp[:_B]


def kernel(user_ids, item_ids, user_table, item_table):
    uids = user_ids.astype(_i32)
    iids = item_ids.astype(_i32)
    return _two_tower(uids, iids, user_table, item_table)


# speedup vs baseline: 6.8718x; 6.8718x over previous
import functools

import jax
import jax.numpy as jnp
from jax import lax
from jax.experimental import pallas as pl
from jax.experimental.pallas import tpu as pltpu
from jax.experimental.pallas import tpu_sc as plsc

_B = 16384
_D = 64
_V = 100000
_NC = 2
_NW = 32
_W = 256
_NWIN_FULL = _V // _W
_TAIL = _V - _NWIN_FULL * _W
_TAIL_WORKER = _NWIN_FULL % _NW
_TAIL_J = _NWIN_FULL // _NW
_UVI_ROWS = _B + 128
_HALF = _B // 2

_i32 = jnp.int32
_f32 = jnp.float32


def _worker_id():
    return lax.axis_index("s") * _NC + lax.axis_index("c")


def _scan_ids(w, idsv, mlist, lane):
    def body(i, cnts):
        cA, cB = cnts
        idsA = idsv[pl.ds(i * 32, 16)]
        idsB = idsv[pl.ds(i * 32 + 16, 16)]
        kA = lax.shift_right_logical(idsA, 8)
        kB = lax.shift_right_logical(idsB, 8)
        mA = (kA & 31) == w
        mB = (kB & 31) == w
        eA = ((i * 32 + lane) << 12) \
            | (lax.shift_right_logical(kA, 5) << 8) | (idsA & 255)
        eB = ((i * 32 + 16 + lane) << 12) \
            | (lax.shift_right_logical(kB, 5) << 8) | (idsB & 255)
        plsc.store_compressed(mlist.at[pl.ds(cA, 16)], eA, mask=mA)
        plsc.store_compressed(mlist.at[pl.ds(_HALF + cB, 16)], eB, mask=mB)
        return (cA + plsc.all_reduce_population_count(mA)[0],
                cB + plsc.all_reduce_population_count(mB)[0])

    return lax.fori_loop(0, _B // 32, body, (0, 0))


def _collect_window(j, cntA, cntB, mlist, wlist, lane):
    def mk(off, cnt):
        def body(ci, wcnt):
            e = mlist[pl.ds(off + ci * 16, 16)]
            sel = ((lax.shift_right_logical(e, 8) & 15) == j) \
                & ((ci * 16 + lane) < cnt)
            plsc.store_compressed(wlist.at[pl.ds(wcnt, 16)], e, mask=sel)
            return wcnt + plsc.all_reduce_population_count(sel)[0]
        return body

    wcnt = lax.fori_loop(0, lax.shift_right_logical(cntA + 15, 4),
                         mk(0, cntA), 0)
    wcnt = lax.fori_loop(0, lax.shift_right_logical(cntB + 15, 4),
                         mk(_HALF, cntB), wcnt)
    return wcnt


def _init_dummy_rows(bst, base):
    lane = lax.iota(_i32, 16)
    for m in range(8):
        bst[pl.ds(m * 16, 16)] = base + m * 16 + lane


def _tower(tT, tTt, ids, uvi, idsv, mlist, wlist, win, sbuf, bst, fillref,
           wsem):
    w = _worker_id()
    lane = lax.iota(_i32, 16)
    pltpu.sync_copy(ids, idsv)
    cntA, cntB = _scan_ids(w, idsv, mlist, lane)

    fillref[0] = 0
    _init_dummy_rows(bst, _B)

    def flush():
        pltpu.sync_copy(sbuf, uvi.at[bst])
        _init_dummy_rows(bst, _B)
        fillref[0] = 0

    def extract(winref, wcnt):
        def body(ci, carry):
            base = ci * 16
            e = wlist[pl.ds(base, 16)]
            sel = lane < (wcnt - base)
            cols = e & 255
            bs = lax.shift_right_logical(e, 12)

            @pl.when(fillref[0] + 16 > 128)
            def _():
                flush()

            fill = fillref[0]
            slot = fill + plsc.cumsum(sel.astype(_i32)) - 1
            for d in range(_D):
                dv = jnp.full((16,), d, _i32)
                vals = plsc.load_gather(winref, [dv, cols], mask=sel)
                plsc.store_scatter(sbuf, [slot, dv], vals, mask=sel)
            plsc.store_scatter(bst, [slot], bs, mask=sel)
            fillref[0] = fill + plsc.all_reduce_population_count(sel)[0]
            return carry

        nec = lax.shift_right_logical(wcnt + 15, 4)
        lax.fori_loop(0, nec, body, 0)

    nfull = jnp.where(w < _TAIL_WORKER, _TAIL_J + 1, _TAIL_J)

    pltpu.async_copy(tT.at[:, pl.ds(w * _W, _W)], win.at[0], wsem.at[0])

    def win_body(j, carry):
        slot = j & 1

        @pl.when(j + 1 < nfull)
        def _():
            k = w + 32 * (j + 1)
            pltpu.async_copy(tT.at[:, pl.ds(k * _W, _W)], win.at[1 - slot],
                             wsem.at[1 - slot])

        pltpu.make_async_copy(tT.at[:, pl.ds(0, _W)], win.at[slot],
                              wsem.at[slot]).wait()
        wcnt = _collect_window(j, cntA, cntB, mlist, wlist, lane)
        extract(win.at[slot], wcnt)
        return carry

    lax.fori_loop(0, nfull, win_body, 0)

    @pl.when(w == _TAIL_WORKER)
    def _():
        pltpu.sync_copy(tTt, win.at[0])
        wcnt = _collect_window(_TAIL_J, cntA, cntB, mlist, wlist, lane)
        extract(win.at[0], wcnt)

    @pl.when(fillref[0] > 0)
    def _():
        flush()


def _zip_dot(uvi, ivi, out, ubuf, ibuf, ovec, *sems):
    w = _worker_id()
    lane = lax.iota(_i32, 16)
    rows_per_w = _B // _NW
    nch = rows_per_w // 128
    base = w * rows_per_w
    usems, isems = sems[:2], sems[2:]

    def fire(c, slot):
        pltpu.async_copy(uvi.at[pl.ds(base + c * 128, 128)], ubuf.at[slot],
                         usems[slot])
        pltpu.async_copy(ivi.at[pl.ds(base + c * 128, 128)], ibuf.at[slot],
                         isems[slot])

    fire(0, 0)
    for c in range(nch):
        slot = c & 1
        if c + 1 < nch:
            fire(c + 1, 1 - slot)
        pltpu.make_async_copy(uvi.at[pl.ds(0, 128)], ubuf.at[slot],
                              usems[slot]).wait()
        pltpu.make_async_copy(ivi.at[pl.ds(0, 128)], ibuf.at[slot],
                              isems[slot]).wait()
        for g in range(8):
            rows = g * 16 + lane
            accs = [jnp.zeros((16,), _f32) for _ in range(4)]
            for d in range(_D):
                dv = jnp.full((16,), d, _i32)
                uv = plsc.load_gather(ubuf.at[slot], [rows, dv])
                iv = plsc.load_gather(ibuf.at[slot], [rows, dv])
                accs[d % 4] = accs[d % 4] + uv * iv
            ovec[pl.ds(c * 128 + g * 16, 16)] = \
                (accs[0] + accs[1]) + (accs[2] + accs[3])
    pltpu.sync_copy(ovec, out.at[pl.ds(base, rows_per_w)])


@jax.jit
def _two_tower(uids, iids, utab, itab):
    mesh = plsc.VectorSubcoreMesh(core_axis_name="c", subcore_axis_name="s")
    cp = pltpu.CompilerParams(needs_layout_passes=False,
                              use_tc_tiling_on_sc=True)
    zpad = jnp.zeros((_W - _TAIL, _D), _f32)
    utT = utab.T
    itT = itab.T
    utTt = jnp.concatenate([utab[_NWIN_FULL * _W:], zpad]).T
    itTt = jnp.concatenate([itab[_NWIN_FULL * _W:], zpad]).T

    tower = functools.partial(
        pl.kernel, mesh=mesh,
        out_type=jax.ShapeDtypeStruct((_UVI_ROWS, 128), _f32),
        scratch_types=[
            pltpu.VMEM((_B,), _i32),
            pltpu.VMEM((_B,), _i32),
            pltpu.VMEM((_B,), _i32),
            pltpu.VMEM((2, _D, _W), _f32),
            pltpu.VMEM((128, 128), _f32),
            pltpu.VMEM((128,), _i32),
            pltpu.SMEM((1,), _i32),
            pltpu.SemaphoreType.DMA((2,)),
        ],
        compiler_params=cp,
    )(_tower)
    uvi = tower(utT, utTt, uids)
    ivi = tower(itT, itTt, iids)

    zip_dot = functools.partial(
        pl.kernel, mesh=mesh,
        out_type=jax.ShapeDtypeStruct((_B,), _f32),
        scratch_types=[
            pltpu.VMEM((2, 128, 128), _f32),
            pltpu.VMEM((2, 128, 128), _f32),
            pltpu.VMEM((_B // _NW,), _f32),
        ] + [pltpu.SemaphoreType.DMA] * 4,
        compiler_params=cp,
    )(_zip_dot)
    return zip_dot(uvi, ivi)


def kernel(user_ids, item_ids, user_table, item_table):
    uids = user_ids.astype(_i32)
    iids = item_ids.astype(_i32)
    return _two_tower(uids, iids, user_table, item_table)


# speedup vs baseline: 9.9363x; 1.4460x over previous
import functools

import jax
import jax.numpy as jnp
from jax import lax
from jax.experimental import pallas as pl
from jax.experimental.pallas import tpu as pltpu
from jax.experimental.pallas import tpu_sc as plsc

_B = 16384
_D = 64
_V = 100000
_NC = 2
_NW = 32
_W = 256
_NWIN_FULL = _V // _W
_TAIL = _V - _NWIN_FULL * _W
_TAIL_WORKER = _NWIN_FULL % _NW
_TAIL_J = _NWIN_FULL // _NW
_UVI_ROWS = _B + 128
_HALF = _B // 2

_i32 = jnp.int32
_f32 = jnp.float32


def _worker_id():
    return lax.axis_index("s") * _NC + lax.axis_index("c")


def _scan_ids(w, idsv, mlist, lane):
    def body(i, cnts):
        cA, cB = cnts
        idsA = idsv[pl.ds(i * 32, 16)]
        idsB = idsv[pl.ds(i * 32 + 16, 16)]
        kA = lax.shift_right_logical(idsA, 8)
        kB = lax.shift_right_logical(idsB, 8)
        mA = (kA & 31) == w
        mB = (kB & 31) == w
        eA = ((i * 32 + lane) << 12) \
            | (lax.shift_right_logical(kA, 5) << 8) | (idsA & 255)
        eB = ((i * 32 + 16 + lane) << 12) \
            | (lax.shift_right_logical(kB, 5) << 8) | (idsB & 255)
        plsc.store_compressed(mlist.at[pl.ds(cA, 16)], eA, mask=mA)
        plsc.store_compressed(mlist.at[pl.ds(_HALF + cB, 16)], eB, mask=mB)
        return (cA + plsc.all_reduce_population_count(mA)[0],
                cB + plsc.all_reduce_population_count(mB)[0])

    return lax.fori_loop(0, _B // 32, body, (0, 0))


def _collect_window(j, cntA, cntB, mlist, wlist, lane):
    def mk(off, cnt):
        def body(ci, wcnt):
            e = mlist[pl.ds(off + ci * 16, 16)]
            sel = ((lax.shift_right_logical(e, 8) & 15) == j) \
                & ((ci * 16 + lane) < cnt)
            plsc.store_compressed(wlist.at[pl.ds(wcnt, 16)], e, mask=sel)
            return wcnt + plsc.all_reduce_population_count(sel)[0]
        return body

    wcnt = lax.fori_loop(0, lax.shift_right_logical(cntA + 15, 4),
                         mk(0, cntA), 0)
    wcnt = lax.fori_loop(0, lax.shift_right_logical(cntB + 15, 4),
                         mk(_HALF, cntB), wcnt)
    return wcnt


def _init_dummy_rows(bst, base):
    lane = lax.iota(_i32, 16)
    for m in range(8):
        bst[pl.ds(m * 16, 16)] = base + m * 16 + lane


def _tower(tT, tTt, ids, uvi, idsv, mlist, wlist, win, sbuf, bst, fillref,
           wsem):
    w = _worker_id()
    lane = lax.iota(_i32, 16)
    pltpu.sync_copy(ids, idsv)
    cntA, cntB = _scan_ids(w, idsv, mlist, lane)

    fillref[0] = 0
    _init_dummy_rows(bst, _B)

    def flush():
        pltpu.sync_copy(sbuf, uvi.at[bst])
        _init_dummy_rows(bst, _B)
        fillref[0] = 0

    def extract(winref, wcnt):
        def body(ci, carry):
            base = ci * 16
            e = wlist[pl.ds(base, 16)]
            sel = lane < (wcnt - base)
            cols = e & 255
            bs = lax.shift_right_logical(e, 12)

            @pl.when(fillref[0] + 16 > 128)
            def _():
                flush()

            fill = fillref[0]
            slot = fill + plsc.cumsum(sel.astype(_i32)) - 1
            for d in range(_D):
                dv = (lane + d) & (_D - 1)
                vals = plsc.load_gather(winref, [dv, cols], mask=sel)
                plsc.store_scatter(sbuf, [slot, dv], vals, mask=sel)
            plsc.store_scatter(bst, [slot], bs, mask=sel)
            fillref[0] = fill + plsc.all_reduce_population_count(sel)[0]
            return carry

        nec = lax.shift_right_logical(wcnt + 15, 4)
        lax.fori_loop(0, nec, body, 0)

    nfull = jnp.where(w < _TAIL_WORKER, _TAIL_J + 1, _TAIL_J)

    pltpu.async_copy(tT.at[:, pl.ds(w * _W, _W)], win.at[0], wsem.at[0])

    def win_body(j, carry):
        slot = j & 1

        @pl.when(j + 1 < nfull)
        def _():
            k = w + 32 * (j + 1)
            pltpu.async_copy(tT.at[:, pl.ds(k * _W, _W)], win.at[1 - slot],
                             wsem.at[1 - slot])

        pltpu.make_async_copy(tT.at[:, pl.ds(0, _W)], win.at[slot],
                              wsem.at[slot]).wait()
        wcnt = _collect_window(j, cntA, cntB, mlist, wlist, lane)
        extract(win.at[slot], wcnt)
        return carry

    lax.fori_loop(0, nfull, win_body, 0)

    @pl.when(w == _TAIL_WORKER)
    def _():
        pltpu.sync_copy(tTt, win.at[0])
        wcnt = _collect_window(_TAIL_J, cntA, cntB, mlist, wlist, lane)
        extract(win.at[0], wcnt)

    @pl.when(fillref[0] > 0)
    def _():
        flush()


def _zip_dot(uvi, ivi, out, ubuf, ibuf, ovec, *sems):
    w = _worker_id()
    lane = lax.iota(_i32, 16)
    rows_per_w = _B // _NW
    nch = rows_per_w // 128
    base = w * rows_per_w
    usems, isems = sems[:2], sems[2:]

    def fire(c, slot):
        pltpu.async_copy(uvi.at[pl.ds(base + c * 128, 128)], ubuf.at[slot],
                         usems[slot])
        pltpu.async_copy(ivi.at[pl.ds(base + c * 128, 128)], ibuf.at[slot],
                         isems[slot])

    fire(0, 0)
    for c in range(nch):
        slot = c & 1
        if c + 1 < nch:
            fire(c + 1, 1 - slot)
        pltpu.make_async_copy(uvi.at[pl.ds(0, 128)], ubuf.at[slot],
                              usems[slot]).wait()
        pltpu.make_async_copy(ivi.at[pl.ds(0, 128)], ibuf.at[slot],
                              isems[slot]).wait()
        def grp(g, carry, slot=slot, c=c):
            rows = g * 16 + lane
            accs = [jnp.zeros((16,), _f32) for _ in range(4)]
            for d in range(_D):
                dv = (lane + d) & (_D - 1)
                uv = plsc.load_gather(ubuf.at[slot], [rows, dv])
                iv = plsc.load_gather(ibuf.at[slot], [rows, dv])
                accs[d % 4] = accs[d % 4] + uv * iv
            ovec[pl.ds(c * 128 + g * 16, 16)] = \
                (accs[0] + accs[1]) + (accs[2] + accs[3])
            return carry

        lax.fori_loop(0, 8, grp, 0)
    pltpu.sync_copy(ovec, out.at[pl.ds(base, rows_per_w)])


@jax.jit
def _two_tower(uids, iids, utab, itab):
    mesh = plsc.VectorSubcoreMesh(core_axis_name="c", subcore_axis_name="s")
    cp = pltpu.CompilerParams(needs_layout_passes=False,
                              use_tc_tiling_on_sc=True)
    zpad = jnp.zeros((_W - _TAIL, _D), _f32)
    utT = utab.T
    itT = itab.T
    utTt = jnp.concatenate([utab[_NWIN_FULL * _W:], zpad]).T
    itTt = jnp.concatenate([itab[_NWIN_FULL * _W:], zpad]).T

    tower = functools.partial(
        pl.kernel, mesh=mesh,
        out_type=jax.ShapeDtypeStruct((_UVI_ROWS, 128), _f32),
        scratch_types=[
            pltpu.VMEM((_B,), _i32),
            pltpu.VMEM((_B,), _i32),
            pltpu.VMEM((_B,), _i32),
            pltpu.VMEM((2, _D, _W), _f32),
            pltpu.VMEM((128, 128), _f32),
            pltpu.VMEM((128,), _i32),
            pltpu.SMEM((1,), _i32),
            pltpu.SemaphoreType.DMA((2,)),
        ],
        compiler_params=cp,
    )(_tower)
    uvi = tower(utT, utTt, uids)
    ivi = tower(itT, itTt, iids)

    zip_dot = functools.partial(
        pl.kernel, mesh=mesh,
        out_type=jax.ShapeDtypeStruct((_B,), _f32),
        scratch_types=[
            pltpu.VMEM((2, 128, 128), _f32),
            pltpu.VMEM((2, 128, 128), _f32),
            pltpu.VMEM((_B // _NW,), _f32),
        ] + [pltpu.SemaphoreType.DMA] * 4,
        compiler_params=cp,
    )(_zip_dot)
    return zip_dot(uvi, ivi)


def kernel(user_ids, item_ids, user_table, item_table):
    uids = user_ids.astype(_i32)
    iids = item_ids.astype(_i32)
    return _two_tower(uids, iids, user_table, item_table)


# speedup vs baseline: 10.0709x; 1.0135x over previous
import functools

import jax
import jax.numpy as jnp
from jax import lax
from jax.experimental import pallas as pl
from jax.experimental.pallas import tpu as pltpu
from jax.experimental.pallas import tpu_sc as plsc

_B = 16384
_D = 64
_V = 100000
_NC = 2
_NW = 32
_W = 256
_NWIN_FULL = _V // _W
_TAIL = _V - _NWIN_FULL * _W
_TAIL_WORKER = _NWIN_FULL % _NW
_TAIL_J = _NWIN_FULL // _NW
_UVI_ROWS = _B + 128
_HALF = _B // 2

_i32 = jnp.int32
_f32 = jnp.float32


def _worker_id():
    return lax.axis_index("s") * _NC + lax.axis_index("c")


def _scan_ids(w, idsv, mlist, lane):
    def body(i, cnts):
        cA, cB = cnts
        idsA = idsv[pl.ds(i * 32, 16)]
        idsB = idsv[pl.ds(i * 32 + 16, 16)]
        kA = lax.shift_right_logical(idsA, 8)
        kB = lax.shift_right_logical(idsB, 8)
        mA = (kA & 31) == w
        mB = (kB & 31) == w
        eA = ((i * 32 + lane) << 12) \
            | (lax.shift_right_logical(kA, 5) << 8) | (idsA & 255)
        eB = ((i * 32 + 16 + lane) << 12) \
            | (lax.shift_right_logical(kB, 5) << 8) | (idsB & 255)
        plsc.store_compressed(mlist.at[pl.ds(cA, 16)], eA, mask=mA)
        plsc.store_compressed(mlist.at[pl.ds(_HALF + cB, 16)], eB, mask=mB)
        return (cA + plsc.all_reduce_population_count(mA)[0],
                cB + plsc.all_reduce_population_count(mB)[0])

    return lax.fori_loop(0, _B // 32, body, (0, 0))


def _collect_window(j, cntA, cntB, mlist, wlist, lane):
    def mk(off, cnt):
        def body(ci, wcnt):
            e = mlist[pl.ds(off + ci * 16, 16)]
            sel = ((lax.shift_right_logical(e, 8) & 15) == j) \
                & ((ci * 16 + lane) < cnt)
            plsc.store_compressed(wlist.at[pl.ds(wcnt, 16)], e, mask=sel)
            return wcnt + plsc.all_reduce_population_count(sel)[0]
        return body

    wcnt = lax.fori_loop(0, lax.shift_right_logical(cntA + 15, 4),
                         mk(0, cntA), 0)
    wcnt = lax.fori_loop(0, lax.shift_right_logical(cntB + 15, 4),
                         mk(_HALF, cntB), wcnt)
    return wcnt


def _dummy_rows(bst, q, lane):
    for m in range(8):
        plsc.store_scatter(bst, [jnp.full((16,), q, _i32), m * 16 + lane],
                           _B + m * 16 + lane)


def _tower(tT, tTt, ids, uvi, idsv, mlist, wlist, win, sbuf, bst, fillref,
           wsem, fsem):
    bst2 = bst
    w = _worker_id()
    lane = lax.iota(_i32, 16)
    pltpu.sync_copy(ids, idsv)
    cntA, cntB = _scan_ids(w, idsv, mlist, lane)

    fillref[0] = 0
    fillref[1] = 0
    _dummy_rows(bst, 0, lane)

    def flush(fsem):
        fc = fillref[1]
        q = fc & 1
        pltpu.async_copy(sbuf.at[pl.ds(q * 128, 128)], uvi.at[bst.at[q]],
                         fsem.at[q])
        fillref[1] = fc + 1
        nq = 1 - q

        @pl.when(fc >= 1)
        def _():
            pltpu.make_async_copy(sbuf.at[pl.ds(0, 128)],
                                  uvi.at[bst.at[nq]], fsem.at[nq]).wait()

        _dummy_rows(bst, nq, lane)
        fillref[0] = 0

    def extract(winref, wcnt, fsem):
        def body(ci, carry):
            base = ci * 16
            e = wlist[pl.ds(base, 16)]
            sel = lane < (wcnt - base)
            cols = e & 255
            bs = lax.shift_right_logical(e, 12)

            @pl.when(fillref[0] + 16 > 128)
            def _():
                flush(fsem)

            fill = fillref[0]
            q16 = (fillref[1] & 1) * 128
            slot = q16 + fill + plsc.cumsum(sel.astype(_i32)) - 1
            for d in range(_D):
                dv = (lane + d) & (_D - 1)
                vals = plsc.load_gather(winref, [dv, cols], mask=sel)
                plsc.store_scatter(sbuf, [slot, dv], vals, mask=sel)
            plsc.store_scatter(bst2, [lax.shift_right_logical(slot, 7),
                                      slot & 127], bs, mask=sel)
            fillref[0] = fill + plsc.all_reduce_population_count(sel)[0]
            return carry

        nec = lax.shift_right_logical(wcnt + 15, 4)
        lax.fori_loop(0, nec, body, 0)

    nfull = jnp.where(w < _TAIL_WORKER, _TAIL_J + 1, _TAIL_J)

    pltpu.async_copy(tT.at[:, pl.ds(w * _W, _W)], win.at[0], wsem.at[0])

    def win_body(j, carry):
        slot = j & 1

        @pl.when(j + 1 < nfull)
        def _():
            k = w + 32 * (j + 1)
            pltpu.async_copy(tT.at[:, pl.ds(k * _W, _W)], win.at[1 - slot],
                             wsem.at[1 - slot])

        pltpu.make_async_copy(tT.at[:, pl.ds(0, _W)], win.at[slot],
                              wsem.at[slot]).wait()
        wcnt = _collect_window(j, cntA, cntB, mlist, wlist, lane)
        extract(win.at[slot], wcnt, fsem)
        return carry

    lax.fori_loop(0, nfull, win_body, 0)

    @pl.when(w == _TAIL_WORKER)
    def _():
        pltpu.sync_copy(tTt, win.at[0])
        wcnt = _collect_window(_TAIL_J, cntA, cntB, mlist, wlist, lane)
        extract(win.at[0], wcnt, fsem)

    @pl.when(fillref[0] > 0)
    def _():
        flush(fsem)

    fc = fillref[1]

    @pl.when(fc >= 1)
    def _():
        q = (fc - 1) & 1
        pltpu.make_async_copy(sbuf.at[pl.ds(0, 128)], uvi.at[bst.at[q]],
                              fsem.at[q]).wait()


def _zip_dot(uvi, ivi, out, ubuf, ibuf, ovec, *sems):
    w = _worker_id()
    lane = lax.iota(_i32, 16)
    rows_per_w = _B // _NW
    nch = rows_per_w // 128
    base = w * rows_per_w
    usems, isems = sems[:2], sems[2:]

    def fire(c, slot):
        pltpu.async_copy(uvi.at[pl.ds(base + c * 128, 128)], ubuf.at[slot],
                         usems[slot])
        pltpu.async_copy(ivi.at[pl.ds(base + c * 128, 128)], ibuf.at[slot],
                         isems[slot])

    fire(0, 0)
    for c in range(nch):
        slot = c & 1
        if c + 1 < nch:
            fire(c + 1, 1 - slot)
        pltpu.make_async_copy(uvi.at[pl.ds(0, 128)], ubuf.at[slot],
                              usems[slot]).wait()
        pltpu.make_async_copy(ivi.at[pl.ds(0, 128)], ibuf.at[slot],
                              isems[slot]).wait()
        def grp(g, carry, slot=slot, c=c):
            rows = g * 16 + lane
            accs = [jnp.zeros((16,), _f32) for _ in range(4)]
            for d in range(_D):
                dv = (lane + d) & (_D - 1)
                uv = plsc.load_gather(ubuf.at[slot], [rows, dv])
                iv = plsc.load_gather(ibuf.at[slot], [rows, dv])
                accs[d % 4] = accs[d % 4] + uv * iv
            ovec[pl.ds(c * 128 + g * 16, 16)] = \
                (accs[0] + accs[1]) + (accs[2] + accs[3])
            return carry

        lax.fori_loop(0, 8, grp, 0)
    pltpu.sync_copy(ovec, out.at[pl.ds(base, rows_per_w)])


@jax.jit
def _two_tower(uids, iids, utab, itab):
    mesh = plsc.VectorSubcoreMesh(core_axis_name="c", subcore_axis_name="s")
    cp = pltpu.CompilerParams(needs_layout_passes=False,
                              use_tc_tiling_on_sc=True)
    zpad = jnp.zeros((_W - _TAIL, _D), _f32)
    utT = utab.T
    itT = itab.T
    utTt = jnp.concatenate([utab[_NWIN_FULL * _W:], zpad]).T
    itTt = jnp.concatenate([itab[_NWIN_FULL * _W:], zpad]).T

    tower = functools.partial(
        pl.kernel, mesh=mesh,
        out_type=jax.ShapeDtypeStruct((_UVI_ROWS, 128), _f32),
        scratch_types=[
            pltpu.VMEM((_B,), _i32),
            pltpu.VMEM((_B,), _i32),
            pltpu.VMEM((_B,), _i32),
            pltpu.VMEM((2, _D, _W), _f32),
            pltpu.VMEM((256, 128), _f32),
            pltpu.VMEM((2, 128), _i32),
            pltpu.SMEM((2,), _i32),
            pltpu.SemaphoreType.DMA((2,)),
            pltpu.SemaphoreType.DMA((2,)),
        ],
        compiler_params=cp,
    )(_tower)
    uvi = tower(utT, utTt, uids)
    ivi = tower(itT, itTt, iids)

    zip_dot = functools.partial(
        pl.kernel, mesh=mesh,
        out_type=jax.ShapeDtypeStruct((_B,), _f32),
        scratch_types=[
            pltpu.VMEM((2, 128, 128), _f32),
            pltpu.VMEM((2, 128, 128), _f32),
            pltpu.VMEM((_B // _NW,), _f32),
        ] + [pltpu.SemaphoreType.DMA] * 4,
        compiler_params=cp,
    )(_zip_dot)
    return zip_dot(uvi, ivi)


def kernel(user_ids, item_ids, user_table, item_table):
    uids = user_ids.astype(_i32)
    iids = item_ids.astype(_i32)
    return _two_tower(uids, iids, user_table, item_table)
